# baseline (speedup 1.0000x reference)
import jax
import jax.numpy as jnp
from jax import lax
from jax.experimental import pallas as pl
from jax.experimental.pallas import tpu as pltpu

N_LAYERS = 12
BEAM = 3
TOPK = 3
VOCAB = 100000
HIST = 20
NEG = -3.4e38
NB = 54
LAG = 27


def _body(logits_ref, save_id_ref, rp_ref, prev_ref, pen_ref, *refs):
    kv_refs = refs[:N_LAYERS]
    out_refs = refs[N_LAYERS:2 * N_LAYERS]
    (tbi_ref, nsi_ref, rp_out_ref, tbp_ref, mli_ref,
     cand_v, cand_i, ring, in_sems, out_sems) = refs[2 * N_LAYERS:]

    x = logits_ref[...] * rp_ref[...]
    m = jnp.max(x, axis=1, keepdims=True)
    lse = jnp.log(jnp.sum(jnp.exp(x - m), axis=1, keepdims=True))
    lg = x - m - lse

    iota = lax.broadcasted_iota(jnp.int32, (BEAM, VOCAB), 1)
    cur = lg
    for k in range(TOPK):
        mx = jnp.max(cur, axis=1, keepdims=True)
        am = jnp.min(jnp.where(cur == mx, iota, VOCAB), axis=1,
                     keepdims=True)
        mxp = mx + prev_ref[...]
        for r in range(BEAM):
            cand_v[r * TOPK + k] = mxp[r, 0]
            cand_i[r * TOPK + k] = am[r, 0]
        if k < TOPK - 1:
            cur = jnp.where(iota == am, NEG, cur)

    b_sel = []
    t_sel = []
    v_sel = []
    for j in range(BEAM):
        bv = cand_v[0]
        bc = jnp.int32(0)
        for c in range(1, BEAM * TOPK):
            take = cand_v[c] > bv
            bv = jnp.where(take, cand_v[c], bv)
            bc = jnp.where(take, jnp.int32(c), bc)
        cand_v[bc] = NEG
        b_sel.append(bc // TOPK)
        t_sel.append(cand_i[bc])
        v_sel.append(bv)

    b = b_sel
    nh = 12
    units = N_LAYERS * BEAM * nh
    f = [jnp.int32(0),
         jnp.where(b[1] == b[0], jnp.int32(0), jnp.int32(1)),
         jnp.where(b[2] == b[0], jnp.int32(0),
                   jnp.where(b[2] == b[1], jnp.int32(1), jnp.int32(2)))]
    gh = [None] * units
    oh = [None] * units
    waited = [False] * units

    def wait_gh(u):
        if u < 0 or waited[u] or gh[u] is None:
            return
        ent = gh[u]
        if isinstance(ent, tuple):
            cond, cp = ent

            @pl.when(cond)
            def _():
                cp.wait()
        else:
            ent.wait()
        waited[u] = True

    def start_out(u):
        l, r = divmod(u, BEAM * nh)
        j, h = divmod(r, nh)
        cp = pltpu.make_async_copy(ring.at[u % NB],
                                   out_refs[l].at[j, h],
                                   out_sems.at[u % NB])
        cp.start()
        oh[u] = cp

    for u in range(units):
        l, r = divmod(u, BEAM * nh)
        j, h = divmod(r, nh)
        if u >= NB:
            oh[u - NB].wait()
        if j == 0:
            cp = pltpu.make_async_copy(kv_refs[l].at[b[j], h],
                                       ring.at[u % NB],
                                       in_sems.at[u % NB])
            cp.start(priority=u % 2)
            gh[u] = cp
        else:
            is_dup = f[j] < j
            not_dup = jnp.logical_not(is_dup)
            src_slot = jnp.int32((u - (j - f[j]) * nh) % NB)
            for back in (nh, 2 * nh):
                if j * nh >= back:
                    wait_gh(u - back)
            cp = pltpu.make_async_copy(kv_refs[l].at[b[j], h],
                                       ring.at[u % NB],
                                       in_sems.at[u % NB])

            @pl.when(not_dup)
            def _():
                cp.start(priority=u % 2)

            @pl.when(is_dup)
            def _():
                ring[pl.ds(u % NB, 1)] = ring[pl.ds(src_slot, 1)]
            gh[u] = (not_dup, cp)
        if u >= LAG:
            wait_gh(u - LAG)
            start_out(u - LAG)

    riota1 = lax.broadcasted_iota(jnp.int32, (BEAM, 1), 0)
    tbi_col = jnp.where(riota1 == 0, t_sel[0],
                        jnp.where(riota1 == 1, t_sel[1], t_sel[2]))
    tbi_ref[...] = tbi_col
    tbp_ref[...] = jnp.where(riota1 == 0, v_sel[0],
                             jnp.where(riota1 == 1, v_sel[1], v_sel[2]))
    riota20 = lax.broadcasted_iota(jnp.int32, (BEAM, HIST), 0)
    sid = jnp.where(riota20 == 0, save_id_ref[pl.ds(b[0], 1), :],
                    jnp.where(riota20 == 1, save_id_ref[pl.ds(b[1], 1), :],
                              save_id_ref[pl.ds(b[2], 1), :]))
    nsi_ref[...] = jnp.concatenate([sid, tbi_col], axis=1)

    mli_ref[...] = tbi_col[0:1, :]
    col_iota = lax.broadcasted_iota(jnp.int32, (1, VOCAB), 1)
    pen = jnp.reshape(pen_ref[...], (1, 1))
    for j in range(BEAM):
        row = rp_ref[pl.ds(b[j], 1), :]
        row = jnp.where(col_iota == t_sel[j], row * pen, row)
        rp_out_ref[pl.ds(j, 1), :] = row

    for u in range(units - LAG, units):
        wait_gh(u)
        start_out(u)
    for u in range(units - NB, units):
        oh[u].wait()


@jax.jit
def _run(kvs, logits, save_id, repeat_penality, previous_prob, penality_value):
    kv_shape = kvs[0].shape
    kvs = [jnp.swapaxes(kv, 2, 3) for kv in kvs]
    kt_shape = kvs[0].shape
    out_shape = (
        [jax.ShapeDtypeStruct(kt_shape, jnp.float32) for _ in range(N_LAYERS)]
        + [
            jax.ShapeDtypeStruct((BEAM, 1), jnp.int32),
            jax.ShapeDtypeStruct((BEAM, HIST + 1), jnp.int32),
            jax.ShapeDtypeStruct((BEAM, VOCAB), jnp.float32),
            jax.ShapeDtypeStruct((BEAM, 1), jnp.float32),
            jax.ShapeDtypeStruct((1, 1), jnp.int32),
        ]
    )
    vmem = pl.BlockSpec(memory_space=pltpu.MemorySpace.VMEM)
    smem = pl.BlockSpec(memory_space=pltpu.SMEM)
    hbm = pl.BlockSpec(memory_space=pl.ANY)
    outs = pl.pallas_call(
        _body,
        out_shape=out_shape,
        compiler_params=pltpu.CompilerParams(skip_device_barrier=True),
        in_specs=[vmem, vmem, vmem, vmem, vmem] + [hbm] * N_LAYERS,
        out_specs=[hbm] * N_LAYERS + [vmem, vmem, vmem, vmem, vmem],
        scratch_shapes=[
            pltpu.SMEM((BEAM * TOPK,), jnp.float32),
            pltpu.SMEM((BEAM * TOPK,), jnp.int32),
            pltpu.VMEM((NB, kt_shape[2], kt_shape[3]), jnp.float32),
            pltpu.SemaphoreType.DMA((NB,)),
            pltpu.SemaphoreType.DMA((NB,)),
        ],
    )(logits, save_id, repeat_penality, previous_prob, penality_value, *kvs)
    save_kv = [jnp.swapaxes(o, 2, 3) for o in outs[:N_LAYERS]]
    tbi, nsi, rp_out, tbp, mli = outs[N_LAYERS:]
    return (*save_kv, tbi, nsi, rp_out, tbp, mli.reshape(1))


def kernel(kv_0, kv_1, kv_2, kv_3, kv_4, kv_5, kv_6, kv_7, kv_8, kv_9,
           kv_10, kv_11, logits, save_id, repeat_penality, previous_prob,
           penality_value, beam_size, topK):
    kvs = (kv_0, kv_1, kv_2, kv_3, kv_4, kv_5, kv_6, kv_7, kv_8, kv_9,
           kv_10, kv_11)
    return _run(kvs, logits, save_id, repeat_penality, previous_prob,
                penality_value)


# speedup vs baseline: 1.0035x; 1.0035x over previous
import jax
import jax.numpy as jnp
from jax import lax
from jax.experimental import pallas as pl
from jax.experimental.pallas import tpu as pltpu

N_LAYERS = 12
BEAM = 3
TOPK = 3
VOCAB = 100000
HIST = 20
NEG = -3.4e38
NB = 40
LAG = 20


def _body(logits_ref, save_id_ref, rp_ref, prev_ref, pen_ref, *refs):
    kv_refs = refs[:N_LAYERS]
    out_refs = refs[N_LAYERS:2 * N_LAYERS]
    (tbi_ref, nsi_ref, rp_out_ref, tbp_ref, mli_ref,
     cand_v, cand_i, ring, in_sems, out_sems) = refs[2 * N_LAYERS:]

    x = logits_ref[...] * rp_ref[...]
    m = jnp.max(x, axis=1, keepdims=True)
    lse = jnp.log(jnp.sum(jnp.exp(x - m), axis=1, keepdims=True))
    lg = x - m - lse

    iota = lax.broadcasted_iota(jnp.int32, (BEAM, VOCAB), 1)
    cur = lg
    for k in range(TOPK):
        mx = jnp.max(cur, axis=1, keepdims=True)
        am = jnp.min(jnp.where(cur == mx, iota, VOCAB), axis=1,
                     keepdims=True)
        mxp = mx + prev_ref[...]
        for r in range(BEAM):
            cand_v[r * TOPK + k] = mxp[r, 0]
            cand_i[r * TOPK + k] = am[r, 0]
        if k < TOPK - 1:
            cur = jnp.where(iota == am, NEG, cur)

    b_sel = []
    t_sel = []
    v_sel = []
    for j in range(BEAM):
        bv = cand_v[0]
        bc = jnp.int32(0)
        for c in range(1, BEAM * TOPK):
            take = cand_v[c] > bv
            bv = jnp.where(take, cand_v[c], bv)
            bc = jnp.where(take, jnp.int32(c), bc)
        cand_v[bc] = NEG
        b_sel.append(bc // TOPK)
        t_sel.append(cand_i[bc])
        v_sel.append(bv)

    b = b_sel
    nh = 12
    units = N_LAYERS * BEAM * nh
    f = [jnp.int32(0),
         jnp.where(b[1] == b[0], jnp.int32(0), jnp.int32(1)),
         jnp.where(b[2] == b[0], jnp.int32(0),
                   jnp.where(b[2] == b[1], jnp.int32(1), jnp.int32(2)))]
    gh = [None] * units
    oh = [None] * units
    waited = [False] * units

    def wait_gh(u):
        if u < 0 or waited[u] or gh[u] is None:
            return
        ent = gh[u]
        if isinstance(ent, tuple):
            cond, cp = ent

            @pl.when(cond)
            def _():
                cp.wait()
        else:
            ent.wait()
        waited[u] = True

    def start_out(u):
        l, r = divmod(u, BEAM * nh)
        j, h = divmod(r, nh)
        cp = pltpu.make_async_copy(ring.at[u % NB],
                                   out_refs[l].at[j, h],
                                   out_sems.at[u % NB])
        cp.start()
        oh[u] = cp

    for u in range(units):
        l, r = divmod(u, BEAM * nh)
        j, h = divmod(r, nh)
        if u >= NB:
            oh[u - NB].wait()
        if j == 0:
            cp = pltpu.make_async_copy(kv_refs[l].at[b[j], h],
                                       ring.at[u % NB],
                                       in_sems.at[u % NB])
            cp.start(priority=u % 2)
            gh[u] = cp
        else:
            is_dup = f[j] < j
            not_dup = jnp.logical_not(is_dup)
            src_slot = jnp.int32((u - (j - f[j]) * nh) % NB)
            for back in (nh, 2 * nh):
                if j * nh >= back:
                    wait_gh(u - back)
            cp = pltpu.make_async_copy(kv_refs[l].at[b[j], h],
                                       ring.at[u % NB],
                                       in_sems.at[u % NB])

            @pl.when(not_dup)
            def _():
                cp.start(priority=u % 2)

            @pl.when(is_dup)
            def _():
                ring[pl.ds(u % NB, 1)] = ring[pl.ds(src_slot, 1)]
            gh[u] = (not_dup, cp)
        if u >= LAG:
            wait_gh(u - LAG)
            start_out(u - LAG)

    riota1 = lax.broadcasted_iota(jnp.int32, (BEAM, 1), 0)
    tbi_col = jnp.where(riota1 == 0, t_sel[0],
                        jnp.where(riota1 == 1, t_sel[1], t_sel[2]))
    tbi_ref[...] = tbi_col
    tbp_ref[...] = jnp.where(riota1 == 0, v_sel[0],
                             jnp.where(riota1 == 1, v_sel[1], v_sel[2]))
    riota20 = lax.broadcasted_iota(jnp.int32, (BEAM, HIST), 0)
    sid = jnp.where(riota20 == 0, save_id_ref[pl.ds(b[0], 1), :],
                    jnp.where(riota20 == 1, save_id_ref[pl.ds(b[1], 1), :],
                              save_id_ref[pl.ds(b[2], 1), :]))
    nsi_ref[...] = jnp.concatenate([sid, tbi_col], axis=1)

    mli_ref[...] = tbi_col[0:1, :]
    col_iota = lax.broadcasted_iota(jnp.int32, (1, VOCAB), 1)
    pen = jnp.reshape(pen_ref[...], (1, 1))
    for j in range(BEAM):
        row = rp_ref[pl.ds(b[j], 1), :]
        row = jnp.where(col_iota == t_sel[j], row * pen, row)
        rp_out_ref[pl.ds(j, 1), :] = row

    for u in range(units - LAG, units):
        wait_gh(u)
        start_out(u)
    for u in range(units - NB, units):
        oh[u].wait()


@jax.jit
def _run(kvs, logits, save_id, repeat_penality, previous_prob, penality_value):
    kv_shape = kvs[0].shape
    kvs = [jnp.swapaxes(kv, 2, 3) for kv in kvs]
    kt_shape = kvs[0].shape
    out_shape = (
        [jax.ShapeDtypeStruct(kt_shape, jnp.float32) for _ in range(N_LAYERS)]
        + [
            jax.ShapeDtypeStruct((BEAM, 1), jnp.int32),
            jax.ShapeDtypeStruct((BEAM, HIST + 1), jnp.int32),
            jax.ShapeDtypeStruct((BEAM, VOCAB), jnp.float32),
            jax.ShapeDtypeStruct((BEAM, 1), jnp.float32),
            jax.ShapeDtypeStruct((1, 1), jnp.int32),
        ]
    )
    vmem = pl.BlockSpec(memory_space=pltpu.MemorySpace.VMEM)
    smem = pl.BlockSpec(memory_space=pltpu.SMEM)
    hbm = pl.BlockSpec(memory_space=pl.ANY)
    outs = pl.pallas_call(
        _body,
        out_shape=out_shape,
        compiler_params=pltpu.CompilerParams(skip_device_barrier=True),
        in_specs=[vmem, vmem, vmem, vmem, vmem] + [hbm] * N_LAYERS,
        out_specs=[hbm] * N_LAYERS + [vmem, vmem, vmem, vmem, vmem],
        scratch_shapes=[
            pltpu.SMEM((BEAM * TOPK,), jnp.float32),
            pltpu.SMEM((BEAM * TOPK,), jnp.int32),
            pltpu.VMEM((NB, kt_shape[2], kt_shape[3]), jnp.float32),
            pltpu.SemaphoreType.DMA((NB,)),
            pltpu.SemaphoreType.DMA((NB,)),
        ],
    )(logits, save_id, repeat_penality, previous_prob, penality_value, *kvs)
    save_kv = [jnp.swapaxes(o, 2, 3) for o in outs[:N_LAYERS]]
    tbi, nsi, rp_out, tbp, mli = outs[N_LAYERS:]
    return (*save_kv, tbi, nsi, rp_out, tbp, mli.reshape(1))


def kernel(kv_0, kv_1, kv_2, kv_3, kv_4, kv_5, kv_6, kv_7, kv_8, kv_9,
           kv_10, kv_11, logits, save_id, repeat_penality, previous_prob,
           penality_value, beam_size, topK):
    kvs = (kv_0, kv_1, kv_2, kv_3, kv_4, kv_5, kv_6, kv_7, kv_8, kv_9,
           kv_10, kv_11)
    return _run(kvs, logits, save_id, repeat_penality, previous_prob,
                penality_value)


# speedup vs baseline: 1.0071x; 1.0035x over previous
import jax
import jax.numpy as jnp
from jax import lax
from jax.experimental import pallas as pl
from jax.experimental.pallas import tpu as pltpu

N_LAYERS = 12
BEAM = 3
TOPK = 3
VOCAB = 100000
HIST = 20
NEG = -3.4e38
NB = 40
LAG = 20


def _body(logits_ref, save_id_ref, rp_ref, prev_ref, pen_ref, *refs):
    kv_refs = refs[:N_LAYERS]
    out_refs = refs[N_LAYERS:2 * N_LAYERS]
    (tbi_ref, nsi_ref, rp_out_ref, tbp_ref, mli_ref,
     cand_v, cand_i, ring, in_sems, out_sems) = refs[2 * N_LAYERS:]

    x = logits_ref[...] * rp_ref[...]
    m = jnp.max(x, axis=1, keepdims=True)
    lse = jnp.log(jnp.sum(jnp.exp(x - m), axis=1, keepdims=True))
    lg = x - m - lse

    iota = lax.broadcasted_iota(jnp.int32, (BEAM, VOCAB), 1)
    cur = lg
    for k in range(TOPK):
        mx = jnp.max(cur, axis=1, keepdims=True)
        am = jnp.min(jnp.where(cur == mx, iota, VOCAB), axis=1,
                     keepdims=True)
        mxp = mx + prev_ref[...]
        for r in range(BEAM):
            cand_v[r * TOPK + k] = mxp[r, 0]
            cand_i[r * TOPK + k] = am[r, 0]
        if k < TOPK - 1:
            cur = jnp.where(iota == am, NEG, cur)

    b_sel = []
    t_sel = []
    v_sel = []
    for j in range(BEAM):
        bv = cand_v[0]
        bc = jnp.int32(0)
        for c in range(1, BEAM * TOPK):
            take = cand_v[c] > bv
            bv = jnp.where(take, cand_v[c], bv)
            bc = jnp.where(take, jnp.int32(c), bc)
        cand_v[bc] = NEG
        b_sel.append(bc // TOPK)
        t_sel.append(cand_i[bc])
        v_sel.append(bv)

    b = b_sel
    nh = 12
    units = N_LAYERS * BEAM * nh
    f = [jnp.int32(0),
         jnp.where(b[1] == b[0], jnp.int32(0), jnp.int32(1)),
         jnp.where(b[2] == b[0], jnp.int32(0),
                   jnp.where(b[2] == b[1], jnp.int32(1), jnp.int32(2)))]
    gh = [None] * units
    oh = [None] * units
    waited = [False] * units

    def wait_gh(u):
        if u < 0 or waited[u] or gh[u] is None:
            return
        ent = gh[u]
        if isinstance(ent, tuple):
            cond, cp = ent

            @pl.when(cond)
            def _():
                cp.wait()
        else:
            ent.wait()
        waited[u] = True

    def start_out(u):
        l, r = divmod(u, BEAM * nh)
        j, h = divmod(r, nh)
        cp = pltpu.make_async_copy(ring.at[u % NB],
                                   out_refs[l].at[j, h],
                                   out_sems.at[u % NB])
        cp.start()
        oh[u] = cp

    for u in range(units):
        l, r = divmod(u, BEAM * nh)
        j, h = divmod(r, nh)
        if u >= NB:
            oh[u - NB].wait()
        if j == 0:
            cp = pltpu.make_async_copy(kv_refs[l].at[b[j], h],
                                       ring.at[u % NB],
                                       in_sems.at[u % NB])
            cp.start(priority=u % 2)
            gh[u] = cp
        else:
            is_dup = f[j] < j
            not_dup = jnp.logical_not(is_dup)
            src_slot = jnp.int32((u - (j - f[j]) * nh) % NB)
            for back in (nh, 2 * nh):
                if j * nh >= back:
                    wait_gh(u - back)
            cp = pltpu.make_async_copy(kv_refs[l].at[b[j], h],
                                       ring.at[u % NB],
                                       in_sems.at[u % NB])

            @pl.when(not_dup)
            def _():
                cp.start(priority=u % 2)

            @pl.when(is_dup)
            def _():
                ring[pl.ds(u % NB, 1)] = ring[pl.ds(src_slot, 1)]
            gh[u] = (not_dup, cp)
        if u >= LAG:
            wait_gh(u - LAG)
            start_out(u - LAG)

    riota1 = lax.broadcasted_iota(jnp.int32, (BEAM, 1), 0)
    tbi_col = jnp.where(riota1 == 0, t_sel[0],
                        jnp.where(riota1 == 1, t_sel[1], t_sel[2]))
    tbi_ref[...] = tbi_col
    tbp_ref[...] = jnp.where(riota1 == 0, v_sel[0],
                             jnp.where(riota1 == 1, v_sel[1], v_sel[2]))
    riota20 = lax.broadcasted_iota(jnp.int32, (BEAM, HIST), 0)
    sid = jnp.where(riota20 == 0, save_id_ref[pl.ds(b[0], 1), :],
                    jnp.where(riota20 == 1, save_id_ref[pl.ds(b[1], 1), :],
                              save_id_ref[pl.ds(b[2], 1), :]))
    nsi_ref[...] = jnp.concatenate([sid, tbi_col], axis=1)

    mli_ref[...] = tbi_col[0:1, :]
    col_iota = lax.broadcasted_iota(jnp.int32, (1, VOCAB), 1)
    pen = jnp.reshape(pen_ref[...], (1, 1))
    for j in range(BEAM):
        row = rp_ref[pl.ds(b[j], 1), :]
        row = jnp.where(col_iota == t_sel[j], row * pen, row)
        rp_out_ref[pl.ds(j, 1), :] = row

    for u in range(units - LAG, units):
        wait_gh(u)
        start_out(u)
    for u in range(units - NB, units):
        oh[u].wait()


@jax.jit
def _run(kvs, logits, save_id, repeat_penality, previous_prob, penality_value):
    kv_shape = kvs[0].shape
    kvs = [jnp.swapaxes(kv, 2, 3) for kv in kvs]
    kt_shape = kvs[0].shape
    out_shape = (
        [jax.ShapeDtypeStruct(kt_shape, jnp.float32) for _ in range(N_LAYERS)]
        + [
            jax.ShapeDtypeStruct((BEAM, 1), jnp.int32),
            jax.ShapeDtypeStruct((BEAM, HIST + 1), jnp.int32),
            jax.ShapeDtypeStruct((BEAM, VOCAB), jnp.float32),
            jax.ShapeDtypeStruct((BEAM, 1), jnp.float32),
            jax.ShapeDtypeStruct((1, 1), jnp.int32),
        ]
    )
    vmem = pl.BlockSpec(memory_space=pltpu.MemorySpace.VMEM)
    hbm = pl.BlockSpec(memory_space=pl.ANY)
    outs = pl.pallas_call(
        _body,
        out_shape=out_shape,
        compiler_params=pltpu.CompilerParams(skip_device_barrier=True),
        in_specs=[vmem, vmem, vmem, vmem, vmem] + [hbm] * N_LAYERS,
        out_specs=[hbm] * N_LAYERS + [vmem, vmem, vmem, vmem, vmem],
        scratch_shapes=[
            pltpu.SMEM((BEAM * TOPK,), jnp.float32),
            pltpu.SMEM((BEAM * TOPK,), jnp.int32),
            pltpu.VMEM((NB, kt_shape[2], kt_shape[3]), jnp.float32),
            pltpu.SemaphoreType.DMA((NB,)),
            pltpu.SemaphoreType.DMA((NB,)),
        ],
    )(logits, save_id, repeat_penality, previous_prob, penality_value, *kvs)
    save_kv = [jnp.swapaxes(o, 2, 3) for o in outs[:N_LAYERS]]
    tbi, nsi, rp_out, tbp, mli = outs[N_LAYERS:]
    return (*save_kv, tbi, nsi, rp_out, tbp, mli.reshape(1))


def kernel(kv_0, kv_1, kv_2, kv_3, kv_4, kv_5, kv_6, kv_7, kv_8, kv_9,
           kv_10, kv_11, logits, save_id, repeat_penality, previous_prob,
           penality_value, beam_size, topK):
    kvs = (kv_0, kv_1, kv_2, kv_3, kv_4, kv_5, kv_6, kv_7, kv_8, kv_9,
           kv_10, kv_11)
    return _run(kvs, logits, save_id, repeat_penality, previous_prob,
                penality_value)
